# baseline (speedup 1.0000x reference)
import functools

import jax
import jax.numpy as jnp
from jax import lax
from jax.experimental import pallas as pl
from jax.experimental.pallas import tpu as pltpu
from jax.experimental.pallas import tpu_sc as plsc

LV, B, N = 4, 1024, 8192
BS = 512
R = LV * B
NC, NS, LN = 2, 16, 16
NW = NC * NS
RPW = R // NW
VPR = N // LN
NB = 64
NB2 = 64
SCALE = float(NB * NB2)
NGRP = NB // LN


def _tree16(load):
    vs = [load(c) for c in range(16)]
    while len(vs) > 1:
        vs = [vs[i] + vs[i + 1] for i in range(0, len(vs), 2)]
    return vs[0]


def _sc_body(scores_hbm, masks_hbm, out_hbm, sb, mb, hc, hfc, hfs, res,
             tmp, tmpi, sems, semm):
    wid = lax.axis_index("s") * NC + lax.axis_index("c")
    lane = lax.iota(jnp.int32, 16)
    ones16 = jnp.ones((16,), jnp.float32)
    z16 = jnp.zeros((16,), jnp.float32)

    def shuf(v, idx):
        ref = tmpi if v.dtype == jnp.int32 else tmp
        ref[pl.ds(0, 16)] = v
        return plsc.load_gather(ref, [idx])

    def splat_sum(v):
        for step in (1, 2, 4, 8):
            v = v + shuf(v, lane ^ step)
        return v

    def splat_max(v):
        for step in (1, 2, 4, 8):
            v = jnp.maximum(v, shuf(v, lane ^ step))
        return v

    def prefix_incl(v):
        for step in (1, 2, 4, 8):
            sh = shuf(v, jnp.maximum(lane - step, 0))
            v = v + jnp.where(lane >= step, sh, 0.0)
        return v

    def start_row(r, p):
        row = wid * RPW + r
        pltpu.async_copy(scores_hbm.at[row], sb[p], sems[p])
        pltpu.async_copy(masks_hbm.at[row], mb[p], semm[p])

    def compute_row(r, p):
        row = wid * RPW + r
        sbuf, mbuf = sb[p], mb[p]

        def z_body(i):
            hc[pl.ds(i * 16, 16)] = z16
            hfc[pl.ds(i * 16, 16)] = z16
            hfs[pl.ds(i * 16, 16)] = z16

        plsc.parallel_loop(0, NB, unroll=4)(z_body)

        pltpu.make_async_copy(masks_hbm.at[row], mbuf, semm[p]).wait()
        pltpu.make_async_copy(scores_hbm.at[row], sbuf, sems[p]).wait()

        def p1(j, acc):
            x = sbuf[pl.ds(j * 16, 16)]
            bkt = (x * float(NB)).astype(jnp.int32)
            plsc.addupdate_scatter(hc, [lane * NB + bkt], ones16)
            return acc + mbuf[pl.ds(j * 16, 16)]

        macc = plsc.parallel_loop(0, VPR, unroll=8, carry=z16)(p1)
        t10 = splat_sum(macc) * 0.1
        tif = t10.astype(jnp.int32).astype(jnp.float32)
        kf = jnp.where(t10 > tif, tif + 1.0, tif)

        def scan_top(h, nb):
            best = jnp.zeros((16,), jnp.int32)
            carry = z16
            for g in range(NGRP - 1, -1, -1):
                tot = _tree16(lambda c: h[pl.ds(c * nb + g * 16, 16)])
                cs = prefix_incl(tot)
                total = shuf(cs, lane * 0 + 15)
                suf = total - cs + tot + carry
                cand = jnp.where(suf >= kf, g * 16 + lane, 0)
                best = jnp.maximum(best, cand)
                carry = carry + total
            return splat_max(best)

        bstar = scan_top(hc, NB)
        b0 = bstar.astype(jnp.float32) * (1.0 / NB)

        def p2(j):
            x = sbuf[pl.ds(j * 16, 16)]
            msk = x >= b0
            f = ((x - b0) * SCALE).astype(jnp.int32)
            f = jnp.minimum(jnp.maximum(f, 0), NB2 - 1)
            idx2 = lane * NB2 + f
            plsc.addupdate_scatter(hfc, [idx2], ones16, mask=msk)
            plsc.addupdate_scatter(hfs, [idx2], x, mask=msk)

        plsc.parallel_loop(0, VPR, unroll=8)(p2)

        fstar = scan_top(hfc, NB2)
        t = b0 + fstar.astype(jnp.float32) * (1.0 / SCALE)

        cacc = z16
        sacc = z16
        for g in range(NGRP):
            totc = _tree16(lambda c: hfc[pl.ds(c * NB2 + g * 16, 16)])
            tots = _tree16(lambda c: hfs[pl.ds(c * NB2 + g * 16, 16)])
            m = (g * 16 + lane) >= fstar
            cacc = cacc + jnp.where(m, totc, 0.0)
            sacc = sacc + jnp.where(m, tots, 0.0)
        cge = splat_sum(cacc)
        sge = splat_sum(sacc)

        topk = sge - (cge - kf) * t
        pval = topk / kf
        plsc.store_scatter(res, [lane * 0 + r], pval, mask=lane < 1)

    start_row(0, 0)
    start_row(1, 1)

    def loop_body(i, _):
        r = i * 2
        compute_row(r, 0)

        @pl.when(r + 2 < RPW)
        def _():
            start_row(r + 2, 0)

        compute_row(r + 1, 1)

        @pl.when(r + 3 < RPW)
        def _():
            start_row(r + 3, 1)

        return 0

    lax.fori_loop(0, RPW // 2, loop_body, 0)
    pltpu.sync_copy(res, out_hbm.at[pl.ds(wid * RPW, RPW)])


_sc_rows = functools.partial(
    pl.kernel,
    out_type=jax.ShapeDtypeStruct((R,), jnp.float32),
    mesh=plsc.VectorSubcoreMesh(core_axis_name="c", subcore_axis_name="s"),
    compiler_params=pltpu.CompilerParams(needs_layout_passes=False),
    scratch_types=[
        [pltpu.VMEM((N,), jnp.float32)] * 2,
        [pltpu.VMEM((N,), jnp.float32)] * 2,
        pltpu.VMEM((LN * NB,), jnp.float32),
        pltpu.VMEM((LN * NB2,), jnp.float32),
        pltpu.VMEM((LN * NB2,), jnp.float32),
        pltpu.VMEM((RPW,), jnp.float32),
        pltpu.VMEM((16,), jnp.float32),
        pltpu.VMEM((16,), jnp.int32),
        [pltpu.SemaphoreType.DMA] * 2,
        [pltpu.SemaphoreType.DMA] * 2,
    ],
)(_sc_body)


def _bce_body(px_ref, out_ref):
    p4 = px_ref[...]
    inp = jnp.mean(p4, axis=0, keepdims=True)
    inp = jnp.minimum(inp, 1.0 - 1e-7)
    b_idx = jax.lax.broadcasted_iota(jnp.int32, (1, B), 1)
    target = (b_idx >= BS).astype(jnp.float32)
    log_p = jnp.maximum(jnp.log(inp), -100.0)
    log_1mp = jnp.maximum(jnp.log(1.0 - inp), -100.0)
    out_ref[0, 0] = -jnp.sum(target * log_p + (1.0 - target) * log_1mp) / B


_bce_call = pl.pallas_call(
    _bce_body,
    out_specs=pl.BlockSpec(memory_space=pltpu.SMEM),
    out_shape=jax.ShapeDtypeStruct((1, 1), jnp.float32),
)


def kernel(scores, masks):
    px = _sc_rows(scores.reshape(R, N), masks.reshape(R, N))
    loss = _bce_call(px.reshape(LV, B))
    return loss[0, 0]


# speedup vs baseline: 1.4551x; 1.4551x over previous
import functools

import jax
import jax.numpy as jnp
from jax import lax
from jax.experimental import pallas as pl
from jax.experimental.pallas import tpu as pltpu
from jax.experimental.pallas import tpu_sc as plsc

LV, B, N = 4, 1024, 8192
BS = 512
R = LV * B
NC, NS, LN = 2, 16, 16
NW = NC * NS
RT = 2560
RSC = R - RT
RPW = RSC // NW
BRT = 64
TC_ITERS = 12
VPR = N // LN
NB = 64
NB2 = 64
SCALE = float(NB * NB2)
NGRP = NB // LN


def _tree16(load):
    vs = [load(c) for c in range(16)]
    while len(vs) > 1:
        vs = [vs[i] + vs[i + 1] for i in range(0, len(vs), 2)]
    return vs[0]


def _sc_body(scores_hbm, masks_hbm, out_hbm, sb, mb, hc, hfc, hfs, res,
             tmp, tmpi, sems, semm):
    wid = lax.axis_index("s") * NC + lax.axis_index("c")
    lane = lax.iota(jnp.int32, 16)
    ones16 = jnp.ones((16,), jnp.float32)
    z16 = jnp.zeros((16,), jnp.float32)

    def shuf(v, idx):
        ref = tmpi if v.dtype == jnp.int32 else tmp
        ref[pl.ds(0, 16)] = v
        return plsc.load_gather(ref, [idx])

    def splat_sum(v):
        for step in (1, 2, 4, 8):
            v = v + shuf(v, lane ^ step)
        return v

    def splat_max(v):
        for step in (1, 2, 4, 8):
            v = jnp.maximum(v, shuf(v, lane ^ step))
        return v

    def prefix_incl(v):
        for step in (1, 2, 4, 8):
            sh = shuf(v, jnp.maximum(lane - step, 0))
            v = v + jnp.where(lane >= step, sh, 0.0)
        return v

    def start_row(r, p):
        row = RT + wid * RPW + r
        pltpu.async_copy(scores_hbm.at[row], sb[p], sems[p])
        pltpu.async_copy(masks_hbm.at[row], mb[p], semm[p])

    def compute_row(r, p):
        row = RT + wid * RPW + r
        sbuf, mbuf = sb[p], mb[p]

        def z_body(i):
            hc[pl.ds(i * 16, 16)] = z16
            hfc[pl.ds(i * 16, 16)] = z16
            hfs[pl.ds(i * 16, 16)] = z16

        plsc.parallel_loop(0, NB, unroll=4)(z_body)

        pltpu.make_async_copy(masks_hbm.at[row], mbuf, semm[p]).wait()

        def k_body(j, acc):
            v = [mbuf[pl.ds(j * 16 + u * 16, 16)] for u in range(8)]
            s = ((v[0] + v[1]) + (v[2] + v[3])) + \
                ((v[4] + v[5]) + (v[6] + v[7]))
            return acc + s

        macc = plsc.parallel_loop(0, VPR, step=8, carry=z16)(k_body)
        t10 = splat_sum(macc) * 0.1
        tif = t10.astype(jnp.int32).astype(jnp.float32)
        kf = jnp.where(t10 > tif, tif + 1.0, tif)

        pltpu.make_async_copy(scores_hbm.at[row], sbuf, sems[p]).wait()

        def p1(j):
            x = sbuf[pl.ds(j * 16, 16)]
            bkt = (x * float(NB)).astype(jnp.int32)
            plsc.addupdate_scatter(hc, [lane * NB + bkt], ones16)

        plsc.parallel_loop(0, VPR, unroll=8)(p1)

        def scan_top(h, nb):
            best = jnp.zeros((16,), jnp.int32)
            carry = z16
            for g in range(NGRP - 1, -1, -1):
                tot = _tree16(lambda c: h[pl.ds(c * nb + g * 16, 16)])
                cs = prefix_incl(tot)
                total = shuf(cs, lane * 0 + 15)
                suf = total - cs + tot + carry
                cand = jnp.where(suf >= kf, g * 16 + lane, 0)
                best = jnp.maximum(best, cand)
                carry = carry + total
            return splat_max(best)

        bstar = scan_top(hc, NB)
        b0 = bstar.astype(jnp.float32) * (1.0 / NB)

        def p2(j):
            x = sbuf[pl.ds(j * 16, 16)]
            msk = x >= b0
            f = ((x - b0) * SCALE).astype(jnp.int32)
            f = jnp.minimum(jnp.maximum(f, 0), NB2 - 1)
            idx2 = lane * NB2 + f
            plsc.addupdate_scatter(hfc, [idx2], ones16, mask=msk)
            plsc.addupdate_scatter(hfs, [idx2], x, mask=msk)

        plsc.parallel_loop(0, VPR, unroll=8)(p2)

        fstar = scan_top(hfc, NB2)
        t = b0 + fstar.astype(jnp.float32) * (1.0 / SCALE)

        cacc = z16
        sacc = z16
        for g in range(NGRP):
            totc = _tree16(lambda c: hfc[pl.ds(c * NB2 + g * 16, 16)])
            tots = _tree16(lambda c: hfs[pl.ds(c * NB2 + g * 16, 16)])
            m = (g * 16 + lane) >= fstar
            cacc = cacc + jnp.where(m, totc, 0.0)
            sacc = sacc + jnp.where(m, tots, 0.0)
        cge = splat_sum(cacc)
        sge = splat_sum(sacc)

        topk = sge - (cge - kf) * t
        pval = topk / kf
        plsc.store_scatter(res, [lane * 0 + r], pval, mask=lane < 1)

    start_row(0, 0)
    start_row(1, 1)

    def loop_body(i, _):
        r = i * 2
        compute_row(r, 0)

        @pl.when(r + 2 < RPW)
        def _():
            start_row(r + 2, 0)

        compute_row(r + 1, 1)

        @pl.when(r + 3 < RPW)
        def _():
            start_row(r + 3, 1)

        return 0

    lax.fori_loop(0, RPW // 2, loop_body, 0)
    pltpu.sync_copy(res, out_hbm.at[pl.ds(wid * RPW, RPW)])


_sc_rows = functools.partial(
    pl.kernel,
    out_type=jax.ShapeDtypeStruct((RSC,), jnp.float32),
    mesh=plsc.VectorSubcoreMesh(core_axis_name="c", subcore_axis_name="s"),
    compiler_params=pltpu.CompilerParams(needs_layout_passes=False),
    scratch_types=[
        [pltpu.VMEM((N,), jnp.float32)] * 2,
        [pltpu.VMEM((N,), jnp.float32)] * 2,
        pltpu.VMEM((LN * NB,), jnp.float32),
        pltpu.VMEM((LN * NB2,), jnp.float32),
        pltpu.VMEM((LN * NB2,), jnp.float32),
        pltpu.VMEM((RPW,), jnp.float32),
        pltpu.VMEM((16,), jnp.float32),
        pltpu.VMEM((16,), jnp.int32),
        [pltpu.SemaphoreType.DMA] * 2,
        [pltpu.SemaphoreType.DMA] * 2,
    ],
)(_sc_body)


def _tc_body(s_ref, m_ref, o_ref):
    x = s_ref[...]
    m = m_ref[...]
    ms = jnp.sum(m, axis=-1, keepdims=True) * 0.1
    kf = jnp.ceil(ms)
    lo = jnp.zeros((BRT, 1), jnp.float32)
    hi = jnp.ones((BRT, 1), jnp.float32)
    for _ in range(TC_ITERS):
        mid = 0.5 * (lo + hi)
        cnt = jnp.sum((x >= mid).astype(jnp.float32), axis=-1, keepdims=True)
        pred = cnt >= kf
        lo = jnp.where(pred, mid, lo)
        hi = jnp.where(pred, hi, mid)
    t = lo
    ge = (x >= t).astype(jnp.float32)
    c = jnp.sum(ge, axis=-1, keepdims=True)
    s = jnp.sum(x * ge, axis=-1, keepdims=True)
    o_ref[...] = (s - (c - kf) * t) / kf


_tc_call = pl.pallas_call(
    _tc_body,
    grid=(RT // BRT,),
    in_specs=[
        pl.BlockSpec((BRT, N), lambda i: (i, 0)),
        pl.BlockSpec((BRT, N), lambda i: (i, 0)),
    ],
    out_specs=pl.BlockSpec((BRT, 1), lambda i: (i, 0)),
    out_shape=jax.ShapeDtypeStruct((RT, 1), jnp.float32),
)


def _bce_body(px_ref, out_ref):
    p4 = px_ref[...]
    inp = jnp.mean(p4, axis=0, keepdims=True)
    inp = jnp.minimum(inp, 1.0 - 1e-7)
    b_idx = jax.lax.broadcasted_iota(jnp.int32, (1, B), 1)
    target = (b_idx >= BS).astype(jnp.float32)
    log_p = jnp.maximum(jnp.log(inp), -100.0)
    log_1mp = jnp.maximum(jnp.log(1.0 - inp), -100.0)
    out_ref[0, 0] = -jnp.sum(target * log_p + (1.0 - target) * log_1mp) / B


_bce_call = pl.pallas_call(
    _bce_body,
    out_specs=pl.BlockSpec(memory_space=pltpu.SMEM),
    out_shape=jax.ShapeDtypeStruct((1, 1), jnp.float32),
)


def kernel(scores, masks):
    s2 = scores.reshape(R, N)
    m2 = masks.reshape(R, N)
    px_sc = _sc_rows(s2, m2)
    px_tc = _tc_call(s2[:RT], m2[:RT])[:, 0]
    px = jnp.concatenate([px_tc, px_sc]).reshape(LV, B)
    loss = _bce_call(px)
    return loss[0, 0]


# speedup vs baseline: 1.4559x; 1.0006x over previous
import functools

import jax
import jax.numpy as jnp
from jax import lax
from jax.experimental import pallas as pl
from jax.experimental.pallas import tpu as pltpu
from jax.experimental.pallas import tpu_sc as plsc

LV, B, N = 4, 1024, 8192
BS = 512
R = LV * B
NC, NS, LN = 2, 16, 16
NW = NC * NS
RT = 2560
RSC = R - RT
RPW = RSC // NW
BRT = 64
TC_ITERS = 12
VPR = N // LN
NB = 64
NB2 = 64
SCALE = float(NB * NB2)
NGRP = NB // LN


def _tree16(load):
    vs = [load(c) for c in range(16)]
    while len(vs) > 1:
        vs = [vs[i] + vs[i + 1] for i in range(0, len(vs), 2)]
    return vs[0]


def _sc_body(scores_hbm, masks_hbm, out_hbm, sb, mb, hc, hfc, hfs, res,
             tmp, tmpi, sems, semm):
    wid = lax.axis_index("s") * NC + lax.axis_index("c")
    lane = lax.iota(jnp.int32, 16)
    ones16 = jnp.ones((16,), jnp.float32)
    z16 = jnp.zeros((16,), jnp.float32)

    def shuf(v, idx):
        ref = tmpi if v.dtype == jnp.int32 else tmp
        ref[pl.ds(0, 16)] = v
        return plsc.load_gather(ref, [idx])

    def splat_sum(v):
        for step in (1, 2, 4, 8):
            v = v + shuf(v, lane ^ step)
        return v

    def splat_max(v):
        for step in (1, 2, 4, 8):
            v = jnp.maximum(v, shuf(v, lane ^ step))
        return v

    def prefix_incl(v):
        for step in (1, 2, 4, 8):
            sh = shuf(v, jnp.maximum(lane - step, 0))
            v = v + jnp.where(lane >= step, sh, 0.0)
        return v

    def start_row(r, p):
        row = RT + wid * RPW + r
        pltpu.async_copy(scores_hbm.at[row], sb[p], sems[p])
        pltpu.async_copy(masks_hbm.at[row], mb[p], semm[p])

    def compute_row(r, p):
        row = RT + wid * RPW + r
        sbuf, mbuf = sb[p], mb[p]

        def z_body(i):
            hc[pl.ds(i * 16, 16)] = z16
            hfc[pl.ds(i * 16, 16)] = z16
            hfs[pl.ds(i * 16, 16)] = z16

        plsc.parallel_loop(0, NB, unroll=4)(z_body)

        pltpu.make_async_copy(masks_hbm.at[row], mbuf, semm[p]).wait()

        def k_body(j, acc):
            v = [mbuf[pl.ds(j * 16 + u * 16, 16)] for u in range(8)]
            s = ((v[0] + v[1]) + (v[2] + v[3])) + \
                ((v[4] + v[5]) + (v[6] + v[7]))
            return acc + s

        macc = plsc.parallel_loop(0, VPR, step=8, carry=z16)(k_body)
        t10 = splat_sum(macc) * 0.1
        tif = t10.astype(jnp.int32).astype(jnp.float32)
        kf = jnp.where(t10 > tif, tif + 1.0, tif)

        pltpu.make_async_copy(scores_hbm.at[row], sbuf, sems[p]).wait()

        def p1(j):
            x = sbuf[pl.ds(j * 16, 16)]
            bkt = (x * float(NB)).astype(jnp.int32)
            plsc.addupdate_scatter(hc, [lane * NB + bkt], ones16)

        plsc.parallel_loop(0, VPR, unroll=8)(p1)

        def scan_top(h, nb):
            best = jnp.zeros((16,), jnp.int32)
            carry = z16
            for g in range(NGRP - 1, -1, -1):
                tot = _tree16(lambda c: h[pl.ds(c * nb + g * 16, 16)])
                cs = prefix_incl(tot)
                total = shuf(cs, lane * 0 + 15)
                suf = total - cs + tot + carry
                cand = jnp.where(suf >= kf, g * 16 + lane, 0)
                best = jnp.maximum(best, cand)
                carry = carry + total
            return splat_max(best)

        bstar = scan_top(hc, NB)
        b0 = bstar.astype(jnp.float32) * (1.0 / NB)

        def p2(j):
            x = sbuf[pl.ds(j * 16, 16)]
            msk = x >= b0
            f = ((x - b0) * SCALE).astype(jnp.int32)
            f = jnp.minimum(jnp.maximum(f, 0), NB2 - 1)
            idx2 = lane * NB2 + f
            plsc.addupdate_scatter(hfc, [idx2], ones16, mask=msk)
            plsc.addupdate_scatter(hfs, [idx2], x, mask=msk)

        plsc.parallel_loop(0, VPR, unroll=8)(p2)

        fstar = scan_top(hfc, NB2)
        t = b0 + fstar.astype(jnp.float32) * (1.0 / SCALE)

        cacc = z16
        sacc = z16
        for g in range(NGRP):
            totc = _tree16(lambda c: hfc[pl.ds(c * NB2 + g * 16, 16)])
            tots = _tree16(lambda c: hfs[pl.ds(c * NB2 + g * 16, 16)])
            m = (g * 16 + lane) >= fstar
            cacc = cacc + jnp.where(m, totc, 0.0)
            sacc = sacc + jnp.where(m, tots, 0.0)
        cge = splat_sum(cacc)
        sge = splat_sum(sacc)

        topk = sge - (cge - kf) * t
        pval = topk / kf
        plsc.store_scatter(res, [lane * 0 + r], pval, mask=lane < 1)

    start_row(0, 0)
    start_row(1, 1)

    def loop_body(i, _):
        r = i * 2
        compute_row(r, 0)

        @pl.when(r + 2 < RPW)
        def _():
            start_row(r + 2, 0)

        compute_row(r + 1, 1)

        @pl.when(r + 3 < RPW)
        def _():
            start_row(r + 3, 1)

        return 0

    lax.fori_loop(0, RPW // 2, loop_body, 0)
    pltpu.sync_copy(res, out_hbm.at[pl.ds(wid * RPW, RPW)])


_sc_rows = functools.partial(
    pl.kernel,
    out_type=jax.ShapeDtypeStruct((RSC,), jnp.float32),
    mesh=plsc.VectorSubcoreMesh(core_axis_name="c", subcore_axis_name="s"),
    compiler_params=pltpu.CompilerParams(needs_layout_passes=False),
    scratch_types=[
        [pltpu.VMEM((N,), jnp.float32)] * 2,
        [pltpu.VMEM((N,), jnp.float32)] * 2,
        pltpu.VMEM((LN * NB,), jnp.float32),
        pltpu.VMEM((LN * NB2,), jnp.float32),
        pltpu.VMEM((LN * NB2,), jnp.float32),
        pltpu.VMEM((RPW,), jnp.float32),
        pltpu.VMEM((16,), jnp.float32),
        pltpu.VMEM((16,), jnp.int32),
        [pltpu.SemaphoreType.DMA] * 2,
        [pltpu.SemaphoreType.DMA] * 2,
    ],
)(_sc_body)


def _tc_body(s_ref, m_ref, o_ref):
    x = s_ref[...]
    m = m_ref[...]
    ms = jnp.sum(m, axis=-1, keepdims=True) * 0.1
    kf = jnp.ceil(ms)
    lo = jnp.zeros((BRT, 1), jnp.float32)
    hi = jnp.ones((BRT, 1), jnp.float32)
    for _ in range(TC_ITERS):
        mid = 0.5 * (lo + hi)
        cnt = jnp.sum((x >= mid).astype(jnp.float32), axis=-1, keepdims=True)
        pred = cnt >= kf
        lo = jnp.where(pred, mid, lo)
        hi = jnp.where(pred, hi, mid)
    t = lo
    ge = (x >= t).astype(jnp.float32)
    c = jnp.sum(ge, axis=-1, keepdims=True)
    s = jnp.sum(x * ge, axis=-1, keepdims=True)
    o_ref[...] = (s - (c - kf) * t) / kf


_tc_call = pl.pallas_call(
    _tc_body,
    grid=(RT // BRT,),
    in_specs=[
        pl.BlockSpec((BRT, N), lambda i: (i, 0)),
        pl.BlockSpec((BRT, N), lambda i: (i, 0)),
    ],
    out_specs=pl.BlockSpec((BRT, 1), lambda i: (i, 0)),
    out_shape=jax.ShapeDtypeStruct((RT, 1), jnp.float32),
)


def _bce_body(px_ref, out_ref):
    p4 = px_ref[...]
    inp = jnp.mean(p4, axis=0, keepdims=True)
    inp = jnp.minimum(inp, 1.0 - 1e-7)
    b_idx = jax.lax.broadcasted_iota(jnp.int32, (1, B), 1)
    target = (b_idx >= BS).astype(jnp.float32)
    log_p = jnp.maximum(jnp.log(inp), -100.0)
    log_1mp = jnp.maximum(jnp.log(1.0 - inp), -100.0)
    out_ref[0, 0] = -jnp.sum(target * log_p + (1.0 - target) * log_1mp) / B


_bce_call = pl.pallas_call(
    _bce_body,
    out_specs=pl.BlockSpec(memory_space=pltpu.SMEM),
    out_shape=jax.ShapeDtypeStruct((1, 1), jnp.float32),
)


def kernel(scores, masks):
    s2 = scores.reshape(R, N)
    m2 = masks.reshape(R, N)
    px_tc = _tc_call(s2[:RT], m2[:RT])[:, 0]
    px_sc = _sc_rows(s2, m2)
    px = jnp.concatenate([px_tc, px_sc]).reshape(LV, B)
    loss = _bce_call(px)
    return loss[0, 0]
